# Initial kernel scaffold; baseline (speedup 1.0000x reference)
import jax
import jax.numpy as jnp
from jax.experimental import pallas as pl


def kernel(x, p):
    raise NotImplementedError("write your pallas kernel here")


# speedup vs baseline: 1.5878x; 1.5878x over previous
import functools

import jax
import jax.numpy as jnp
from jax import lax
from jax.experimental import pallas as pl
from jax.experimental.pallas import tpu as pltpu
from jax.experimental.pallas import tpu_sc as plsc

TOPK = 2500
NSORT = 16384


def _score_body(x_ref, p_ref, inorm_ref, y_ref):
    s = jnp.dot(x_ref[...], p_ref[...], preferred_element_type=jnp.float32)
    y_ref[0] = s[:, :, 0] * inorm_ref[0, 0]


def _scores(x, p, inv_norm):
    B, N, F = x.shape
    bn = 2000
    nb = N // bn
    y_t = pl.pallas_call(
        _score_body,
        grid=(nb,),
        in_specs=[
            pl.BlockSpec((B, bn, F), lambda i: (0, i, 0)),
            pl.BlockSpec((F, 1), lambda i: (0, 0)),
            pl.BlockSpec((1, 1), lambda i: (0, 0), memory_space=pltpu.SMEM),
        ],
        out_specs=pl.BlockSpec((1, B, bn), lambda i: (i, 0, 0)),
        out_shape=jax.ShapeDtypeStruct((nb, B, bn), jnp.float32),
    )(x, p, inv_norm)
    return y_t.transpose(1, 0, 2).reshape(B, N)


def _sort_body(y_ref, idx_ref, scale_ref):
    B, n = y_ref.shape
    v = -y_ref[...]
    col = lax.broadcasted_iota(jnp.int32, (B, n), 1)
    row = lax.broadcasted_iota(jnp.int32, (B, n), 0)
    idx = row * 10000 + col

    logn = n.bit_length() - 1
    for k in range(1, logn + 1):
        asc = (col & (1 << k)) == 0

        def substage(jj, carry, k=k, asc=asc):
            v, idx = carry
            d = jnp.int32(1) << (k - 1 - jj)
            is_low = (col & d) == 0
            fwd = (jnp.int32(n) - d).astype(jnp.int32)
            pv = jnp.where(is_low, pltpu.roll(v, fwd, axis=1),
                           pltpu.roll(v, d, axis=1))
            pi = jnp.where(is_low, pltpu.roll(idx, fwd, axis=1),
                           pltpu.roll(idx, d, axis=1))
            le = (v < pv) | ((v == pv) & (idx < pi))
            take_self = le == (is_low == asc)
            return jnp.where(take_self, v, pv), jnp.where(take_self, idx, pi)

        v, idx = lax.fori_loop(0, k, substage, (v, idx))

    topv = -v[:, :TOPK]
    idx_ref[...] = idx[:, :TOPK]
    scale_ref[...] = jax.nn.sigmoid(topv)


def _topk(y_pad):
    B, n = y_pad.shape
    return pl.pallas_call(
        _sort_body,
        out_shape=(
            jax.ShapeDtypeStruct((B, TOPK), jnp.int32),
            jax.ShapeDtypeStruct((B, TOPK), jnp.float32),
        ),
    )(y_pad)


def _make_gather(R, F, rows_per_w, chunk):
    info = plsc.get_sparse_core_info()
    nc, ns = info.num_cores, info.num_subcores
    nchunks = rows_per_w // chunk
    mesh = plsc.VectorSubcoreMesh(core_axis_name="c", subcore_axis_name="s")

    @functools.partial(
        pl.kernel,
        mesh=mesh,
        out_type=jax.ShapeDtypeStruct((R, F), jnp.float32),
        scratch_types=[
            pltpu.VMEM((chunk,), jnp.int32),
            pltpu.VMEM((chunk, F), jnp.float32),
            pltpu.SemaphoreType.DMA,
        ],
    )
    def gather(x_hbm, idx_hbm, out_hbm, idx_v, rows_v, sem):
        wid = lax.axis_index("s") * nc + lax.axis_index("c")
        base = wid * rows_per_w
        for t in range(nchunks):
            b0 = base + t * chunk
            pltpu.sync_copy(idx_hbm.at[pl.ds(b0, chunk)], idx_v)
            pltpu.async_copy(x_hbm.at[idx_v], rows_v, sem).wait()
            pltpu.sync_copy(rows_v, out_hbm.at[pl.ds(b0, chunk)])

    return gather


def _scale_body(g_ref, s_ref, o_ref):
    o_ref[...] = g_ref[...] * s_ref[0][:, :, None]


def _apply_scale(gathered, scale):
    B, K = scale.shape
    F = gathered.shape[-1]
    g3 = gathered.reshape(B, K, F)
    s3 = scale.reshape(B, 1, K)
    return pl.pallas_call(
        _scale_body,
        grid=(B,),
        in_specs=[
            pl.BlockSpec((1, K, F), lambda b: (b, 0, 0)),
            pl.BlockSpec((1, 1, K), lambda b: (b, 0, 0)),
        ],
        out_specs=pl.BlockSpec((1, K, F), lambda b: (b, 0, 0)),
        out_shape=jax.ShapeDtypeStruct((B, K, F), jnp.float32),
    )(g3, s3)


def kernel(x, p):
    B, N, F = x.shape
    inv_norm = (1.0 / jnp.linalg.norm(p)).reshape(1, 1).astype(jnp.float32)

    y = _scores(x, p, inv_norm)
    y_pad = jnp.pad(y, ((0, 0), (0, NSORT - N)), constant_values=-jnp.inf)
    idx, scale = _topk(y_pad)

    R = B * TOPK
    RPAD = 20480
    pad_idx = (jnp.arange(RPAD - R, dtype=jnp.int32) * 64) % (B * N)
    idx_flat = jnp.concatenate([idx.reshape(R), pad_idx])

    gather = _make_gather(RPAD, F, rows_per_w=RPAD // 32, chunk=128)
    gathered = gather(x.reshape(B * N, F), idx_flat)

    return _apply_scale(gathered[:R], scale)


# speedup vs baseline: 1.7374x; 1.0942x over previous
import functools

import jax
import jax.numpy as jnp
from jax import lax
from jax.experimental import pallas as pl
from jax.experimental.pallas import tpu as pltpu
from jax.experimental.pallas import tpu_sc as plsc

TOPK = 2500
NSORT = 16384


def _score_body(x_ref, p_ref, inorm_ref, y_ref):
    s = jnp.dot(x_ref[...], p_ref[...], preferred_element_type=jnp.float32)
    y_ref[0] = s[:, :, 0] * inorm_ref[0, 0]


def _scores(x, p, inv_norm):
    B, N, F = x.shape
    bn = 2000
    nb = N // bn
    y_t = pl.pallas_call(
        _score_body,
        grid=(nb,),
        in_specs=[
            pl.BlockSpec((B, bn, F), lambda i: (0, i, 0)),
            pl.BlockSpec((F, 1), lambda i: (0, 0)),
            pl.BlockSpec((1, 1), lambda i: (0, 0), memory_space=pltpu.SMEM),
        ],
        out_specs=pl.BlockSpec((1, B, bn), lambda i: (i, 0, 0)),
        out_shape=jax.ShapeDtypeStruct((nb, B, bn), jnp.float32),
    )(x, p, inv_norm)
    return y_t.transpose(1, 0, 2).reshape(B, N)


def _cmpex(v, ix, dr, asc, span):
    q = lax.broadcasted_iota(jnp.int32, v.shape, 1)
    is_low = (q & dr) == 0
    fwd = jnp.int32(span) - dr
    pv = jnp.where(is_low, pltpu.roll(v, fwd, axis=1), pltpu.roll(v, dr, axis=1))
    pi = jnp.where(is_low, pltpu.roll(ix, fwd, axis=1), pltpu.roll(ix, dr, axis=1))
    le = (v < pv) | ((v == pv) & (ix < pi))
    take_self = le == (is_low == asc)
    return jnp.where(take_self, v, pv), jnp.where(take_self, ix, pi)


def _sort_body(yt_ref, idx_ref, scale_ref):
    shp = yt_ref.shape
    c1 = lax.broadcasted_iota(jnp.int32, shp, 1)
    r2 = lax.broadcasted_iota(jnp.int32, shp, 2)
    b0 = lax.broadcasted_iota(jnp.int32, shp, 0)
    v = -yt_ref[...]
    ix = (b0 * 10000 + r2 * 128 + c1).astype(jnp.float32)

    for k in range(1, 8):
        asc = ((c1 & (1 << k)) == 0) if k <= 6 else ((r2 & 1) == 0)

        def sub_t(jj, carry, k=k, asc=asc):
            d = jnp.int32(1) << (k - 1 - jj)
            return _cmpex(*carry, d, asc, 128)

        v, ix = lax.fori_loop(0, k, sub_t, (v, ix))

    for k in range(8, 15):
        kb = 1 << (k - 7)
        v = jnp.swapaxes(v, 1, 2)
        ix = jnp.swapaxes(ix, 1, 2)
        asc_n = (c1 & kb) == 0

        def sub_n(jj, carry, k=k, asc_n=asc_n):
            dr = jnp.int32(1) << (k - 8 - jj)
            return _cmpex(*carry, dr, asc_n, 128)

        v, ix = lax.fori_loop(0, k - 7, sub_n, (v, ix))
        v = jnp.swapaxes(v, 1, 2)
        ix = jnp.swapaxes(ix, 1, 2)
        asc_t = (r2 & kb) == 0

        def sub_t2(jj, carry, asc_t=asc_t):
            d = jnp.int32(1) << (6 - jj)
            return _cmpex(*carry, d, asc_t, 128)

        v, ix = lax.fori_loop(0, 7, sub_t2, (v, ix))

    vn = jnp.swapaxes(v, 1, 2)
    ixn = jnp.swapaxes(ix, 1, 2)
    idx_ref[...] = ixn.astype(jnp.int32)
    scale_ref[...] = jax.nn.sigmoid(-vn)


def _topk(y_pad):
    B, n = y_pad.shape
    yt = y_pad.reshape(B, n // 128, 128).transpose(0, 2, 1)
    idx3, scale3 = pl.pallas_call(
        _sort_body,
        out_shape=(
            jax.ShapeDtypeStruct((B, n // 128, 128), jnp.int32),
            jax.ShapeDtypeStruct((B, n // 128, 128), jnp.float32),
        ),
    )(yt)
    return (idx3.reshape(B, n)[:, :TOPK], scale3.reshape(B, n)[:, :TOPK])


def _make_gather(R, F, rows_per_w, chunk):
    info = plsc.get_sparse_core_info()
    nc, ns = info.num_cores, info.num_subcores
    nchunks = rows_per_w // chunk
    mesh = plsc.VectorSubcoreMesh(core_axis_name="c", subcore_axis_name="s")

    @functools.partial(
        pl.kernel,
        mesh=mesh,
        out_type=jax.ShapeDtypeStruct((R, F), jnp.float32),
        scratch_types=[
            pltpu.VMEM((chunk,), jnp.int32),
            pltpu.VMEM((chunk, F), jnp.float32),
            pltpu.SemaphoreType.DMA,
        ],
    )
    def gather(x_hbm, idx_hbm, out_hbm, idx_v, rows_v, sem):
        wid = lax.axis_index("s") * nc + lax.axis_index("c")
        base = wid * rows_per_w
        for t in range(nchunks):
            b0 = base + t * chunk
            pltpu.sync_copy(idx_hbm.at[pl.ds(b0, chunk)], idx_v)
            pltpu.async_copy(x_hbm.at[idx_v], rows_v, sem).wait()
            pltpu.sync_copy(rows_v, out_hbm.at[pl.ds(b0, chunk)])

    return gather


def _scale_body(g_ref, s_ref, o_ref):
    o_ref[...] = g_ref[...] * s_ref[0][:, :, None]


def _apply_scale(gathered, scale):
    B, K = scale.shape
    F = gathered.shape[-1]
    g3 = gathered.reshape(B, K, F)
    s3 = scale.reshape(B, 1, K)
    return pl.pallas_call(
        _scale_body,
        grid=(B,),
        in_specs=[
            pl.BlockSpec((1, K, F), lambda b: (b, 0, 0)),
            pl.BlockSpec((1, 1, K), lambda b: (b, 0, 0)),
        ],
        out_specs=pl.BlockSpec((1, K, F), lambda b: (b, 0, 0)),
        out_shape=jax.ShapeDtypeStruct((B, K, F), jnp.float32),
    )(g3, s3)


def kernel(x, p):
    B, N, F = x.shape
    inv_norm = (1.0 / jnp.linalg.norm(p)).reshape(1, 1).astype(jnp.float32)

    y = _scores(x, p, inv_norm)
    y_pad = jnp.pad(y, ((0, 0), (0, NSORT - N)), constant_values=-jnp.inf)
    idx, scale = _topk(y_pad)

    R = B * TOPK
    RPAD = 20480
    pad_idx = (jnp.arange(RPAD - R, dtype=jnp.int32) * 64) % (B * N)
    idx_flat = jnp.concatenate([idx.reshape(R), pad_idx])

    gather = _make_gather(RPAD, F, rows_per_w=RPAD // 32, chunk=128)
    gathered = gather(x.reshape(B * N, F), idx_flat)

    return _apply_scale(gathered[:R], scale)


# speedup vs baseline: 1.9596x; 1.1279x over previous
import functools

import jax
import jax.numpy as jnp
from jax import lax
from jax.experimental import pallas as pl
from jax.experimental.pallas import tpu as pltpu
from jax.experimental.pallas import tpu_sc as plsc

TOPK = 2500
NSORT = 16384


def _score_body(x_ref, p_ref, inorm_ref, y_ref):
    s = jnp.dot(x_ref[...], p_ref[...], preferred_element_type=jnp.float32)
    y_ref[0] = s[:, :, 0] * inorm_ref[0, 0]


def _scores(x, p, inv_norm):
    B, N, F = x.shape
    bn = 2000
    nb = N // bn
    y_t = pl.pallas_call(
        _score_body,
        grid=(nb,),
        in_specs=[
            pl.BlockSpec((B, bn, F), lambda i: (0, i, 0)),
            pl.BlockSpec((F, 1), lambda i: (0, 0)),
            pl.BlockSpec((1, 1), lambda i: (0, 0), memory_space=pltpu.SMEM),
        ],
        out_specs=pl.BlockSpec((1, B, bn), lambda i: (i, 0, 0)),
        out_shape=jax.ShapeDtypeStruct((nb, B, bn), jnp.float32),
    )(x, p, inv_norm)
    return y_t.transpose(1, 0, 2).reshape(B, N)


def _cmpex(v, ix, dr, asc, span):
    q = lax.broadcasted_iota(jnp.int32, v.shape, 1)
    is_low = (q & dr) == 0
    fwd = jnp.int32(span) - dr
    pv = jnp.where(is_low, pltpu.roll(v, fwd, axis=1), pltpu.roll(v, dr, axis=1))
    pi = jnp.where(is_low, pltpu.roll(ix, fwd, axis=1), pltpu.roll(ix, dr, axis=1))
    le = (v < pv) | ((v == pv) & (ix < pi))
    take_self = le == (is_low == asc)
    return jnp.where(take_self, v, pv), jnp.where(take_self, ix, pi)


def _sort_body(yt_ref, idx_ref, scale_ref):
    shp = yt_ref.shape
    c1 = lax.broadcasted_iota(jnp.int32, shp, 1)
    r2 = lax.broadcasted_iota(jnp.int32, shp, 2)
    b0 = lax.broadcasted_iota(jnp.int32, shp, 0)
    v = -yt_ref[...]
    ix = (b0 * 10000 + r2 * 128 + c1).astype(jnp.float32)

    for k in range(1, 8):
        asc = ((c1 & (1 << k)) == 0) if k <= 6 else ((r2 & 1) == 0)

        def sub_t(jj, carry, k=k, asc=asc):
            d = jnp.int32(1) << (k - 1 - jj)
            return _cmpex(*carry, d, asc, 128)

        v, ix = lax.fori_loop(0, k, sub_t, (v, ix))

    for k in range(8, 15):
        kb = 1 << (k - 7)
        v = jnp.swapaxes(v, 1, 2)
        ix = jnp.swapaxes(ix, 1, 2)
        asc_n = (c1 & kb) == 0

        def sub_n(jj, carry, k=k, asc_n=asc_n):
            dr = jnp.int32(1) << (k - 8 - jj)
            return _cmpex(*carry, dr, asc_n, 128)

        v, ix = lax.fori_loop(0, k - 7, sub_n, (v, ix))
        v = jnp.swapaxes(v, 1, 2)
        ix = jnp.swapaxes(ix, 1, 2)
        asc_t = (r2 & kb) == 0

        def sub_t2(jj, carry, asc_t=asc_t):
            d = jnp.int32(1) << (6 - jj)
            return _cmpex(*carry, d, asc_t, 128)

        v, ix = lax.fori_loop(0, 7, sub_t2, (v, ix))

    vn = jnp.swapaxes(v, 1, 2)
    ixn = jnp.swapaxes(ix, 1, 2)
    idx_ref[...] = ixn.astype(jnp.int32)
    scale_ref[...] = jax.nn.sigmoid(-vn)


def _topk(y_pad):
    B, n = y_pad.shape
    yt = y_pad.reshape(B, n // 128, 128).transpose(0, 2, 1)
    idx3, scale3 = pl.pallas_call(
        _sort_body,
        out_shape=(
            jax.ShapeDtypeStruct((B, n // 128, 128), jnp.int32),
            jax.ShapeDtypeStruct((B, n // 128, 128), jnp.float32),
        ),
    )(yt)
    return (idx3.reshape(B, n)[:, :TOPK], scale3.reshape(B, n)[:, :TOPK])


def _make_gather(R, F, rows_per_w, chunk):
    info = plsc.get_sparse_core_info()
    nc, ns = info.num_cores, info.num_subcores
    nchunks = rows_per_w // chunk
    mesh = plsc.VectorSubcoreMesh(core_axis_name="c", subcore_axis_name="s")

    @functools.partial(
        pl.kernel,
        mesh=mesh,
        out_type=jax.ShapeDtypeStruct((R, F), jnp.float32),
        scratch_types=[
            pltpu.VMEM((chunk,), jnp.int32),
            pltpu.VMEM((chunk, F), jnp.float32),
            pltpu.SemaphoreType.DMA,
        ],
    )
    def gather(x_hbm, idx_hbm, out_hbm, idx_v, rows_v, sem):
        wid = lax.axis_index("s") * nc + lax.axis_index("c")
        base = wid * rows_per_w
        for t in range(nchunks):
            b0 = base + t * chunk
            pltpu.sync_copy(idx_hbm.at[pl.ds(b0, chunk)], idx_v)
            pltpu.async_copy(x_hbm.at[idx_v], rows_v, sem).wait()
            pltpu.sync_copy(rows_v, out_hbm.at[pl.ds(b0, chunk)])

    return gather


def _scale_body(g_ref, s_ref, o_ref):
    o_ref[...] = g_ref[:, :TOPK, :] * s_ref[0][:, :, None]


def _apply_scale(gathered, scale, kpad):
    B, K = scale.shape
    F = gathered.shape[-1]
    g3 = gathered.reshape(B, kpad, F)
    s3 = scale.reshape(B, 1, K)
    return pl.pallas_call(
        _scale_body,
        grid=(B,),
        in_specs=[
            pl.BlockSpec((1, kpad, F), lambda b: (b, 0, 0)),
            pl.BlockSpec((1, 1, K), lambda b: (b, 0, 0)),
        ],
        out_specs=pl.BlockSpec((1, K, F), lambda b: (b, 0, 0)),
        out_shape=jax.ShapeDtypeStruct((B, K, F), jnp.float32),
    )(g3, s3)


def kernel(x, p):
    B, N, F = x.shape
    inv_norm = (1.0 / jnp.linalg.norm(p)).reshape(1, 1).astype(jnp.float32)

    y = _scores(x, p, inv_norm)
    y_pad = jnp.pad(y, ((0, 0), (0, NSORT - N)), constant_values=-jnp.inf)
    idx, scale = _topk(y_pad)

    KPAD = 2560
    pad2d = (jnp.arange(KPAD - TOPK, dtype=jnp.int32) * 128)[None, :] + \
        (jnp.arange(B, dtype=jnp.int32) * 1000)[:, None]
    idx_flat = jnp.concatenate([idx, pad2d], axis=1).reshape(B * KPAD)

    RPAD = B * KPAD
    gather = _make_gather(RPAD, F, rows_per_w=RPAD // 32, chunk=128)
    gathered = gather(x.reshape(B * N, F), idx_flat)

    return _apply_scale(gathered, scale, KPAD)


# speedup vs baseline: 2.0883x; 1.0657x over previous
import functools

import jax
import jax.numpy as jnp
from jax import lax
from jax.experimental import pallas as pl
from jax.experimental.pallas import tpu as pltpu
from jax.experimental.pallas import tpu_sc as plsc

TOPK = 2500
NSORT = 16384


def _score_body(x_ref, p_ref, inorm_ref, y_ref):
    s = jnp.dot(x_ref[...], p_ref[...], preferred_element_type=jnp.float32)
    y_ref[0] = s[:, :, 0] * inorm_ref[0, 0]


def _scores(x, p, inv_norm):
    B, N, F = x.shape
    bn = 2000
    nb = N // bn
    y_t = pl.pallas_call(
        _score_body,
        grid=(nb,),
        in_specs=[
            pl.BlockSpec((B, bn, F), lambda i: (0, i, 0)),
            pl.BlockSpec((F, 1), lambda i: (0, 0)),
            pl.BlockSpec((1, 1), lambda i: (0, 0), memory_space=pltpu.SMEM),
        ],
        out_specs=pl.BlockSpec((1, B, bn), lambda i: (i, 0, 0)),
        out_shape=jax.ShapeDtypeStruct((nb, B, bn), jnp.float32),
    )(x, p, inv_norm)
    return y_t.transpose(1, 0, 2).reshape(B, N)


def _cmpex(w, dr, asc, span):
    q = lax.broadcasted_iota(jnp.int32, w.shape[1:], 1)
    is_low = (q & dr) == 0
    rf = pltpu.roll(w, jnp.int32(span) - dr, axis=2)
    rb = pltpu.roll(rf, (dr + dr) & (span - 1), axis=2)
    pw = jnp.where(is_low[None], rf, rb)
    v, ix, pv, pi = w[0], w[1], pw[0], pw[1]
    le = (v < pv) | ((v == pv) & (ix < pi))
    take_self = le == (is_low == asc)
    return jnp.where(take_self[None], w, pw)


def _sort_body(yt_ref, idx_ref, scale_ref):
    shp = yt_ref.shape
    c1 = lax.broadcasted_iota(jnp.int32, shp, 1)
    r2 = lax.broadcasted_iota(jnp.int32, shp, 2)
    b0 = lax.broadcasted_iota(jnp.int32, shp, 0)
    v = -yt_ref[...]
    ix = (b0 * 10000 + r2 * 128 + c1).astype(jnp.float32)
    w = jnp.stack([v, ix])

    for k in range(1, 8):
        asc = ((c1 & (1 << k)) == 0) if k <= 6 else ((r2 & 1) == 0)

        def sub_t(jj, w, k=k, asc=asc):
            d = jnp.int32(1) << (k - 1 - jj)
            return _cmpex(w, d, asc, 128)

        w = lax.fori_loop(0, k, sub_t, w)

    for k in range(8, 15):
        kb = 1 << (k - 7)
        w = jnp.swapaxes(w, 2, 3)
        asc_n = (c1 & kb) == 0

        def sub_n(jj, w, k=k, asc_n=asc_n):
            dr = jnp.int32(1) << (k - 8 - jj)
            return _cmpex(w, dr, asc_n, 128)

        w = lax.fori_loop(0, k - 7, sub_n, w)
        w = jnp.swapaxes(w, 2, 3)
        asc_t = (r2 & kb) == 0

        def sub_t2(jj, w, asc_t=asc_t):
            d = jnp.int32(1) << (6 - jj)
            return _cmpex(w, d, asc_t, 128)

        w = lax.fori_loop(0, 7, sub_t2, w)

    wn = jnp.swapaxes(w, 2, 3)
    idx_ref[...] = wn[1].astype(jnp.int32)
    scale_ref[...] = jax.nn.sigmoid(-wn[0])


def _topk(y_pad):
    B, n = y_pad.shape
    yt = y_pad.reshape(B, n // 128, 128).transpose(0, 2, 1)
    idx3, scale3 = pl.pallas_call(
        _sort_body,
        out_shape=(
            jax.ShapeDtypeStruct((B, n // 128, 128), jnp.int32),
            jax.ShapeDtypeStruct((B, n // 128, 128), jnp.float32),
        ),
    )(yt)
    return (idx3.reshape(B, n)[:, :TOPK], scale3.reshape(B, n)[:, :TOPK])


def _make_gather(R, F, rows_per_w, chunk):
    info = plsc.get_sparse_core_info()
    nc, ns = info.num_cores, info.num_subcores
    nchunks = rows_per_w // chunk
    mesh = plsc.VectorSubcoreMesh(core_axis_name="c", subcore_axis_name="s")

    @functools.partial(
        pl.kernel,
        mesh=mesh,
        out_type=jax.ShapeDtypeStruct((R, F), jnp.float32),
        scratch_types=[
            pltpu.VMEM((chunk,), jnp.int32),
            pltpu.VMEM((chunk, F), jnp.float32),
            pltpu.SemaphoreType.DMA,
        ],
    )
    def gather(x_hbm, idx_hbm, out_hbm, idx_v, rows_v, sem):
        wid = lax.axis_index("s") * nc + lax.axis_index("c")
        base = wid * rows_per_w
        for t in range(nchunks):
            b0 = base + t * chunk
            pltpu.sync_copy(idx_hbm.at[pl.ds(b0, chunk)], idx_v)
            pltpu.async_copy(x_hbm.at[idx_v], rows_v, sem).wait()
            pltpu.sync_copy(rows_v, out_hbm.at[pl.ds(b0, chunk)])

    return gather


def _scale_body(g_ref, s_ref, o_ref):
    o_ref[...] = g_ref[:, :TOPK, :] * s_ref[0][:, :, None]


def _apply_scale(gathered, scale, kpad):
    B, K = scale.shape
    F = gathered.shape[-1]
    g3 = gathered.reshape(B, kpad, F)
    s3 = scale.reshape(B, 1, K)
    return pl.pallas_call(
        _scale_body,
        grid=(B,),
        in_specs=[
            pl.BlockSpec((1, kpad, F), lambda b: (b, 0, 0)),
            pl.BlockSpec((1, 1, K), lambda b: (b, 0, 0)),
        ],
        out_specs=pl.BlockSpec((1, K, F), lambda b: (b, 0, 0)),
        out_shape=jax.ShapeDtypeStruct((B, K, F), jnp.float32),
    )(g3, s3)


def kernel(x, p):
    B, N, F = x.shape
    inv_norm = (1.0 / jnp.linalg.norm(p)).reshape(1, 1).astype(jnp.float32)

    y = _scores(x, p, inv_norm)
    y_pad = jnp.pad(y, ((0, 0), (0, NSORT - N)), constant_values=-jnp.inf)
    idx, scale = _topk(y_pad)

    KPAD = 2560
    pad2d = (jnp.arange(KPAD - TOPK, dtype=jnp.int32) * 128)[None, :] + \
        (jnp.arange(B, dtype=jnp.int32) * 1000)[:, None]
    idx_flat = jnp.concatenate([idx, pad2d], axis=1).reshape(B * KPAD)

    RPAD = B * KPAD
    gather = _make_gather(RPAD, F, rows_per_w=RPAD // 32, chunk=128)
    gathered = gather(x.reshape(B * N, F), idx_flat)

    return _apply_scale(gathered, scale, KPAD)


# speedup vs baseline: 2.8491x; 1.3643x over previous
import functools

import jax
import jax.numpy as jnp
from jax import lax
from jax.experimental import pallas as pl
from jax.experimental.pallas import tpu as pltpu
from jax.experimental.pallas import tpu_sc as plsc

TOPK = 2500
NSORT = 16384


def _score_body(x_ref, p_ref, inorm_ref, y_ref):
    s = jnp.dot(x_ref[...], p_ref[...], preferred_element_type=jnp.float32)
    y_ref[0] = s[:, :, 0] * inorm_ref[0, 0]


def _scores(x, p, inv_norm):
    B, N, F = x.shape
    bn = 2000
    nb = N // bn
    y_t = pl.pallas_call(
        _score_body,
        grid=(nb,),
        in_specs=[
            pl.BlockSpec((B, bn, F), lambda i: (0, i, 0)),
            pl.BlockSpec((F, 1), lambda i: (0, 0)),
            pl.BlockSpec((1, 1), lambda i: (0, 0), memory_space=pltpu.SMEM),
        ],
        out_specs=pl.BlockSpec((1, B, bn), lambda i: (i, 0, 0)),
        out_shape=jax.ShapeDtypeStruct((nb, B, bn), jnp.float32),
    )(x, p, inv_norm)
    return y_t.transpose(1, 0, 2).reshape(B, N)


def _cx(wb, d, asc, enable):
    q = lax.broadcasted_iota(jnp.int32, (128, 128), 0)
    is_low = (q & d) == 0
    rf = pltpu.roll(wb, 128 - d, axis=1)
    rb = pltpu.roll(rf, (2 * d) % 128, axis=1)
    pwb = jnp.where(is_low[None], rf, rb)
    le = (wb[0] < pwb[0]) | ((wb[0] == pwb[0]) & (wb[1] < pwb[1]))
    take_self = le == (is_low == asc)
    nw = jnp.where(take_self[None], wb, pwb)
    return jnp.where(enable, nw, wb)


def _chain(w_ref, asc2, first_m):
    def body(b, _):
        wb = w_ref[:, b]
        for m in range(6, -1, -1):
            wb = _cx(wb, 1 << m, asc2, jnp.int32(m) <= first_m)
        w_ref[:, b] = wb
        return 0
    lax.fori_loop(0, 8, body, 0)


def _sort_body(yt_ref, idx_ref, scale_ref, w_ref):
    shp = yt_ref.shape
    c1 = lax.broadcasted_iota(jnp.int32, shp, 1)
    r2 = lax.broadcasted_iota(jnp.int32, shp, 2)
    b0 = lax.broadcasted_iota(jnp.int32, shp, 0)
    v = -yt_ref[...]
    ix = (b0 * 10000 + r2 * 128 + c1).astype(jnp.float32)
    w_ref[...] = jnp.stack([v, ix])

    q1 = lax.broadcasted_iota(jnp.int32, (128, 128), 0)
    q2 = lax.broadcasted_iota(jnp.int32, (128, 128), 1)

    def p0_body(k, _):
        m1 = jnp.where(k <= 6, jnp.int32(1) << k, jnp.int32(0))
        m2 = jnp.where(k <= 6, jnp.int32(0), jnp.int32(1))
        asc = ((q1 & m1) | (q2 & m2)) == 0
        _chain(w_ref, asc, k - 1)
        return 0

    lax.fori_loop(1, 8, p0_body, 0)

    def p1_body(k, _):
        kb = jnp.int32(1) << (k - 7)
        w_ref[...] = jnp.swapaxes(w_ref[...], 2, 3)
        _chain(w_ref, (q1 & kb) == 0, k - 8)
        w_ref[...] = jnp.swapaxes(w_ref[...], 2, 3)
        _chain(w_ref, (q2 & kb) == 0, jnp.int32(6))
        return 0

    lax.fori_loop(8, 15, p1_body, 0)

    wn = jnp.swapaxes(w_ref[...], 2, 3)
    idx_ref[...] = wn[1].astype(jnp.int32)
    scale_ref[...] = jax.nn.sigmoid(-wn[0])


def _topk(y_pad):
    B, n = y_pad.shape
    yt = y_pad.reshape(B, n // 128, 128).transpose(0, 2, 1)
    idx3, scale3 = pl.pallas_call(
        _sort_body,
        out_shape=(
            jax.ShapeDtypeStruct((B, n // 128, 128), jnp.int32),
            jax.ShapeDtypeStruct((B, n // 128, 128), jnp.float32),
        ),
        scratch_shapes=[pltpu.VMEM((2, B, 128, 128), jnp.float32)],
    )(yt)
    return (idx3.reshape(B, n)[:, :TOPK], scale3.reshape(B, n)[:, :TOPK])


def _make_gather(R, F, rows_per_w, chunk):
    info = plsc.get_sparse_core_info()
    nc, ns = info.num_cores, info.num_subcores
    nchunks = rows_per_w // chunk
    mesh = plsc.VectorSubcoreMesh(core_axis_name="c", subcore_axis_name="s")

    @functools.partial(
        pl.kernel,
        mesh=mesh,
        out_type=jax.ShapeDtypeStruct((R, F), jnp.float32),
        scratch_types=[
            pltpu.VMEM((chunk,), jnp.int32),
            pltpu.VMEM((chunk, F), jnp.float32),
            pltpu.SemaphoreType.DMA,
        ],
    )
    def gather(x_hbm, idx_hbm, out_hbm, idx_v, rows_v, sem):
        wid = lax.axis_index("s") * nc + lax.axis_index("c")
        base = wid * rows_per_w
        for t in range(nchunks):
            b0 = base + t * chunk
            pltpu.sync_copy(idx_hbm.at[pl.ds(b0, chunk)], idx_v)
            pltpu.async_copy(x_hbm.at[idx_v], rows_v, sem).wait()
            pltpu.sync_copy(rows_v, out_hbm.at[pl.ds(b0, chunk)])

    return gather


def _scale_body(g_ref, s_ref, o_ref):
    o_ref[...] = g_ref[:, :TOPK, :] * s_ref[0][:, :, None]


def _apply_scale(gathered, scale, kpad):
    B, K = scale.shape
    F = gathered.shape[-1]
    g3 = gathered.reshape(B, kpad, F)
    s3 = scale.reshape(B, 1, K)
    return pl.pallas_call(
        _scale_body,
        grid=(B,),
        in_specs=[
            pl.BlockSpec((1, kpad, F), lambda b: (b, 0, 0)),
            pl.BlockSpec((1, 1, K), lambda b: (b, 0, 0)),
        ],
        out_specs=pl.BlockSpec((1, K, F), lambda b: (b, 0, 0)),
        out_shape=jax.ShapeDtypeStruct((B, K, F), jnp.float32),
    )(g3, s3)


def kernel(x, p):
    B, N, F = x.shape
    inv_norm = (1.0 / jnp.linalg.norm(p)).reshape(1, 1).astype(jnp.float32)

    y = _scores(x, p, inv_norm)
    y_pad = jnp.pad(y, ((0, 0), (0, NSORT - N)), constant_values=-jnp.inf)
    idx, scale = _topk(y_pad)

    KPAD = 2560
    pad2d = (jnp.arange(KPAD - TOPK, dtype=jnp.int32) * 128)[None, :] + \
        (jnp.arange(B, dtype=jnp.int32) * 1000)[:, None]
    idx_flat = jnp.concatenate([idx, pad2d], axis=1).reshape(B * KPAD)

    RPAD = B * KPAD
    gather = _make_gather(RPAD, F, rows_per_w=RPAD // 32, chunk=128)
    gathered = gather(x.reshape(B * N, F), idx_flat)

    return _apply_scale(gathered, scale, KPAD)


# speedup vs baseline: 3.1303x; 1.0987x over previous
import functools

import jax
import jax.numpy as jnp
from jax import lax
from jax.experimental import pallas as pl
from jax.experimental.pallas import tpu as pltpu
from jax.experimental.pallas import tpu_sc as plsc

TOPK = 2500
NSORT = 16384


def _score_body(x_ref, p_ref, inorm_ref, y_ref):
    s = jnp.dot(x_ref[...], p_ref[...], preferred_element_type=jnp.float32)
    y_ref[0] = s[:, :, 0] * inorm_ref[0, 0]


def _scores(x, p, inv_norm):
    B, N, F = x.shape
    bn = 2000
    nb = N // bn
    y_t = pl.pallas_call(
        _score_body,
        grid=(nb,),
        in_specs=[
            pl.BlockSpec((B, bn, F), lambda i: (0, i, 0)),
            pl.BlockSpec((F, 1), lambda i: (0, 0)),
            pl.BlockSpec((1, 1), lambda i: (0, 0), memory_space=pltpu.SMEM),
        ],
        out_specs=pl.BlockSpec((1, B, bn), lambda i: (i, 0, 0)),
        out_shape=jax.ShapeDtypeStruct((nb, B, bn), jnp.float32),
    )(x, p, inv_norm)
    return y_t.transpose(1, 0, 2).reshape(B, N)


def _cx(wb, d, asc):
    q = lax.broadcasted_iota(jnp.int32, (128, 128), 0)
    is_low = (q & d) == 0
    rf = pltpu.roll(wb, 128 - d, axis=1)
    rb = pltpu.roll(rf, (2 * d) % 128, axis=1)
    pwb = jnp.where(is_low[None], rf, rb)
    le = (wb[0] < pwb[0]) | ((wb[0] == pwb[0]) & (wb[1] < pwb[1]))
    take_self = le == (is_low == asc)
    return jnp.where(take_self[None], wb, pwb)


def _stage0(wb, q1, q2):
    for k in range(1, 8):
        asc = ((q1 & (1 << k)) == 0) if k <= 6 else ((q2 & 1) == 0)
        for m in range(k - 1, -1, -1):
            wb = _cx(wb, 1 << m, asc)
    return wb


def _stage_k(wb, q1, q2, k):
    kb = 1 << (k - 7)
    wb = jnp.swapaxes(wb, 1, 2)
    asc_n = (q1 & kb) == 0
    for m in range(k - 8, -1, -1):
        wb = _cx(wb, 1 << m, asc_n)
    wb = jnp.swapaxes(wb, 1, 2)
    asc_t = (q2 & kb) == 0
    for m in range(6, -1, -1):
        wb = _cx(wb, 1 << m, asc_t)
    return wb


def _sort_body(yt_ref, idx_ref, scale_ref, w_ref):
    shp = yt_ref.shape
    c1 = lax.broadcasted_iota(jnp.int32, shp, 1)
    r2 = lax.broadcasted_iota(jnp.int32, shp, 2)
    b0 = lax.broadcasted_iota(jnp.int32, shp, 0)
    v = -yt_ref[...]
    ix = (b0 * 10000 + r2 * 128 + c1).astype(jnp.float32)
    w_ref[...] = jnp.stack([v, ix])

    q1 = lax.broadcasted_iota(jnp.int32, (128, 128), 0)
    q2 = lax.broadcasted_iota(jnp.int32, (128, 128), 1)

    def run(fn):
        def body(b, _):
            w_ref[:, b] = fn(w_ref[:, b])
            return 0
        lax.fori_loop(0, 8, body, 0)

    run(lambda wb: _stage0(wb, q1, q2))
    for k in range(8, 15):
        run(lambda wb, k=k: _stage_k(wb, q1, q2, k))

    wn = jnp.swapaxes(w_ref[...], 2, 3)
    idx_ref[...] = wn[1].astype(jnp.int32)
    scale_ref[...] = jax.nn.sigmoid(-wn[0])


def _topk(y_pad):
    B, n = y_pad.shape
    yt = y_pad.reshape(B, n // 128, 128).transpose(0, 2, 1)
    idx3, scale3 = pl.pallas_call(
        _sort_body,
        out_shape=(
            jax.ShapeDtypeStruct((B, n // 128, 128), jnp.int32),
            jax.ShapeDtypeStruct((B, n // 128, 128), jnp.float32),
        ),
        scratch_shapes=[pltpu.VMEM((2, B, 128, 128), jnp.float32)],
    )(yt)
    return (idx3.reshape(B, n)[:, :TOPK], scale3.reshape(B, n)[:, :TOPK])


def _make_gather(R, F, rows_per_w, chunk):
    info = plsc.get_sparse_core_info()
    nc, ns = info.num_cores, info.num_subcores
    nchunks = rows_per_w // chunk
    mesh = plsc.VectorSubcoreMesh(core_axis_name="c", subcore_axis_name="s")

    @functools.partial(
        pl.kernel,
        mesh=mesh,
        out_type=jax.ShapeDtypeStruct((R, F), jnp.float32),
        scratch_types=[
            pltpu.VMEM((chunk,), jnp.int32),
            pltpu.VMEM((chunk, F), jnp.float32),
            pltpu.SemaphoreType.DMA,
        ],
    )
    def gather(x_hbm, idx_hbm, out_hbm, idx_v, rows_v, sem):
        wid = lax.axis_index("s") * nc + lax.axis_index("c")
        base = wid * rows_per_w
        for t in range(nchunks):
            b0 = base + t * chunk
            pltpu.sync_copy(idx_hbm.at[pl.ds(b0, chunk)], idx_v)
            pltpu.async_copy(x_hbm.at[idx_v], rows_v, sem).wait()
            pltpu.sync_copy(rows_v, out_hbm.at[pl.ds(b0, chunk)])

    return gather


def _scale_body(g_ref, s_ref, o_ref):
    o_ref[...] = g_ref[:, :TOPK, :] * s_ref[0][:, :, None]


def _apply_scale(gathered, scale, kpad):
    B, K = scale.shape
    F = gathered.shape[-1]
    g3 = gathered.reshape(B, kpad, F)
    s3 = scale.reshape(B, 1, K)
    return pl.pallas_call(
        _scale_body,
        grid=(B,),
        in_specs=[
            pl.BlockSpec((1, kpad, F), lambda b: (b, 0, 0)),
            pl.BlockSpec((1, 1, K), lambda b: (b, 0, 0)),
        ],
        out_specs=pl.BlockSpec((1, K, F), lambda b: (b, 0, 0)),
        out_shape=jax.ShapeDtypeStruct((B, K, F), jnp.float32),
    )(g3, s3)


def kernel(x, p):
    B, N, F = x.shape
    inv_norm = (1.0 / jnp.linalg.norm(p)).reshape(1, 1).astype(jnp.float32)

    y = _scores(x, p, inv_norm)
    y_pad = jnp.pad(y, ((0, 0), (0, NSORT - N)), constant_values=-jnp.inf)
    idx, scale = _topk(y_pad)

    KPAD = 2560
    pad2d = (jnp.arange(KPAD - TOPK, dtype=jnp.int32) * 128)[None, :] + \
        (jnp.arange(B, dtype=jnp.int32) * 1000)[:, None]
    idx_flat = jnp.concatenate([idx, pad2d], axis=1).reshape(B * KPAD)

    RPAD = B * KPAD
    gather = _make_gather(RPAD, F, rows_per_w=RPAD // 32, chunk=128)
    gathered = gather(x.reshape(B * N, F), idx_flat)

    return _apply_scale(gathered, scale, KPAD)


# speedup vs baseline: 3.1313x; 1.0003x over previous
import functools

import jax
import jax.numpy as jnp
from jax import lax
from jax.experimental import pallas as pl
from jax.experimental.pallas import tpu as pltpu
from jax.experimental.pallas import tpu_sc as plsc

TOPK = 2500
NSORT = 16384


def _score_body(x_ref, p_ref, inorm_ref, y_ref):
    s = jnp.dot(x_ref[...], p_ref[...], preferred_element_type=jnp.float32)
    y_ref[0] = s[:, :, 0] * inorm_ref[0, 0]


def _scores(x, p, inv_norm):
    B, N, F = x.shape
    bn = 2000
    nb = N // bn
    y_t = pl.pallas_call(
        _score_body,
        grid=(nb,),
        in_specs=[
            pl.BlockSpec((B, bn, F), lambda i: (0, i, 0)),
            pl.BlockSpec((F, 1), lambda i: (0, 0)),
            pl.BlockSpec((1, 1), lambda i: (0, 0), memory_space=pltpu.SMEM),
        ],
        out_specs=pl.BlockSpec((1, B, bn), lambda i: (i, 0, 0)),
        out_shape=jax.ShapeDtypeStruct((nb, B, bn), jnp.float32),
    )(x, p, inv_norm)
    return y_t.transpose(1, 0, 2).reshape(B, N)


def _cx(wb, d, asc):
    q = lax.broadcasted_iota(jnp.int32, (128, 128), 0)
    is_low = (q & d) == 0
    rf = pltpu.roll(wb, 128 - d, axis=1)
    rb = pltpu.roll(rf, (2 * d) % 128, axis=1)
    pwb = jnp.where(is_low[None], rf, rb)
    le = (wb[0] < pwb[0]) | ((wb[0] == pwb[0]) & (wb[1] < pwb[1]))
    take_self = le == (is_low == asc)
    return jnp.where(take_self[None], wb, pwb)


def _stage0(wb, q1, q2):
    for k in range(1, 8):
        asc = ((q1 & (1 << k)) == 0) if k <= 6 else ((q2 & 1) == 0)
        for m in range(k - 1, -1, -1):
            wb = _cx(wb, 1 << m, asc)
    return wb


def _stage_k(wb, q1, q2, k):
    kb = 1 << (k - 7)
    wb = jnp.swapaxes(wb, 1, 2)
    asc_n = (q1 & kb) == 0
    for m in range(k - 8, -1, -1):
        wb = _cx(wb, 1 << m, asc_n)
    wb = jnp.swapaxes(wb, 1, 2)
    asc_t = (q2 & kb) == 0
    for m in range(6, -1, -1):
        wb = _cx(wb, 1 << m, asc_t)
    return wb


def _sort_body(yt_ref, idx_ref, scale_ref, w_ref):
    shp = yt_ref.shape
    c1 = lax.broadcasted_iota(jnp.int32, shp, 1)
    r2 = lax.broadcasted_iota(jnp.int32, shp, 2)
    b0 = lax.broadcasted_iota(jnp.int32, shp, 0)
    v = -yt_ref[...]
    ix = (b0 * 10000 + r2 * 128 + c1).astype(jnp.float32)
    w_ref[...] = jnp.stack([v, ix])

    q1 = lax.broadcasted_iota(jnp.int32, (128, 128), 0)
    q2 = lax.broadcasted_iota(jnp.int32, (128, 128), 1)

    def run(fn):
        def body(b, _):
            w_ref[:, b] = fn(w_ref[:, b])
            return 0
        lax.fori_loop(0, 8, body, 0)

    run(lambda wb: _stage0(wb, q1, q2))
    for k in range(8, 15):
        run(lambda wb, k=k: _stage_k(wb, q1, q2, k))

    wn = jnp.swapaxes(w_ref[...], 2, 3)
    idx_ref[...] = wn[1, :, :KROWS, :].astype(jnp.int32)
    scale_ref[...] = jax.nn.sigmoid(-wn[0, :, :KROWS, :])


KROWS = 20


def _topk(y_pad):
    B, n = y_pad.shape
    yt = y_pad.reshape(B, n // 128, 128).transpose(0, 2, 1)
    idx20, scale20 = pl.pallas_call(
        _sort_body,
        out_shape=(
            jax.ShapeDtypeStruct((B, KROWS, 128), jnp.int32),
            jax.ShapeDtypeStruct((B, KROWS, 128), jnp.float32),
        ),
        scratch_shapes=[pltpu.VMEM((2, B, 128, 128), jnp.float32)],
    )(yt)
    return idx20, scale20


def _make_gather(R, F, rows_per_w, chunk):
    info = plsc.get_sparse_core_info()
    nc, ns = info.num_cores, info.num_subcores
    nchunks = rows_per_w // chunk
    mesh = plsc.VectorSubcoreMesh(core_axis_name="c", subcore_axis_name="s")

    @functools.partial(
        pl.kernel,
        mesh=mesh,
        out_type=jax.ShapeDtypeStruct((R, F), jnp.float32),
        scratch_types=[
            pltpu.VMEM((chunk,), jnp.int32),
            pltpu.VMEM((chunk, F), jnp.float32),
            pltpu.SemaphoreType.DMA,
        ],
    )
    def gather(x_hbm, idx_hbm, out_hbm, idx_v, rows_v, sem):
        wid = lax.axis_index("s") * nc + lax.axis_index("c")
        base = wid * rows_per_w
        for t in range(nchunks):
            b0 = base + t * chunk
            pltpu.sync_copy(idx_hbm.at[pl.ds(b0, chunk)], idx_v)
            pltpu.async_copy(x_hbm.at[idx_v], rows_v, sem).wait()
            pltpu.sync_copy(rows_v, out_hbm.at[pl.ds(b0, chunk)])

    return gather


def _scale_body(g_ref, s_ref, o_ref):
    o_ref[...] = g_ref[:, :TOPK, :] * s_ref[0][:, :TOPK, None]


def _apply_scale(gathered, scale, kpad):
    B = scale.shape[0]
    F = gathered.shape[-1]
    g3 = gathered.reshape(B, kpad, F)
    s3 = scale.reshape(B, 1, kpad)
    return pl.pallas_call(
        _scale_body,
        grid=(B,),
        in_specs=[
            pl.BlockSpec((1, kpad, F), lambda b: (b, 0, 0)),
            pl.BlockSpec((1, 1, kpad), lambda b: (b, 0, 0)),
        ],
        out_specs=pl.BlockSpec((1, TOPK, F), lambda b: (b, 0, 0)),
        out_shape=jax.ShapeDtypeStruct((B, TOPK, F), jnp.float32),
    )(g3, s3)


def kernel(x, p):
    B, N, F = x.shape
    inv_norm = (1.0 / jnp.linalg.norm(p)).reshape(1, 1).astype(jnp.float32)

    y = _scores(x, p, inv_norm)
    y_pad = jnp.pad(y, ((0, 0), (0, NSORT - N)), constant_values=-jnp.inf)
    idx20, scale20 = _topk(y_pad)

    KPAD = KROWS * 128
    idx_flat = idx20.reshape(B * KPAD)

    RPAD = B * KPAD
    gather = _make_gather(RPAD, F, rows_per_w=RPAD // 32, chunk=128)
    gathered = gather(x.reshape(B * N, F), idx_flat)

    return _apply_scale(gathered, scale20, KPAD)


# speedup vs baseline: 3.2193x; 1.0281x over previous
import functools

import jax
import jax.numpy as jnp
from jax import lax
from jax.experimental import pallas as pl
from jax.experimental.pallas import tpu as pltpu
from jax.experimental.pallas import tpu_sc as plsc

TOPK = 2500
NSORT = 16384


def _score_body(x_ref, p_ref, inorm_ref, y_ref):
    s = jnp.dot(x_ref[...], p_ref[...], preferred_element_type=jnp.float32)
    y_ref[0] = s[:, :, 0] * inorm_ref[0, 0]


def _scores(x, p, inv_norm):
    B, N, F = x.shape
    bn = 2000
    nb = N // bn
    y_t = pl.pallas_call(
        _score_body,
        grid=(nb,),
        in_specs=[
            pl.BlockSpec((B, bn, F), lambda i: (0, i, 0)),
            pl.BlockSpec((F, 1), lambda i: (0, 0)),
            pl.BlockSpec((1, 1), lambda i: (0, 0), memory_space=pltpu.SMEM),
        ],
        out_specs=pl.BlockSpec((1, B, bn), lambda i: (i, 0, 0)),
        out_shape=jax.ShapeDtypeStruct((nb, B, bn), jnp.float32),
    )(x, p, inv_norm)
    return y_t.transpose(1, 0, 2).reshape(B, N)


def _cx(wb, d, asc):
    q = lax.broadcasted_iota(jnp.int32, (128, 128), 0)
    is_low = (q & d) == 0
    rf = pltpu.roll(wb, 128 - d, axis=1)
    rb = pltpu.roll(rf, (2 * d) % 128, axis=1)
    pwb = jnp.where(is_low[None], rf, rb)
    le = (wb[0] < pwb[0]) | ((wb[0] == pwb[0]) & (wb[1] < pwb[1]))
    take_self = le == (is_low == asc)
    return jnp.where(take_self[None], wb, pwb)


def _stage0(wb, q1, q2):
    for k in range(1, 8):
        asc = ((q1 & (1 << k)) == 0) if k <= 6 else ((q2 & 1) == 0)
        for m in range(k - 1, -1, -1):
            wb = _cx(wb, 1 << m, asc)
    return wb


def _stage_k(wb, q1, q2, k):
    kb = 1 << (k - 7)
    wb = jnp.swapaxes(wb, 1, 2)
    asc_n = (q1 & kb) == 0
    for m in range(k - 8, -1, -1):
        wb = _cx(wb, 1 << m, asc_n)
    wb = jnp.swapaxes(wb, 1, 2)
    asc_t = (q2 & kb) == 0
    for m in range(6, -1, -1):
        wb = _cx(wb, 1 << m, asc_t)
    return wb


def _sort_body(yt_ref, idx_ref, scale_ref, w_ref):
    shp = yt_ref.shape
    c1 = lax.broadcasted_iota(jnp.int32, shp, 1)
    r2 = lax.broadcasted_iota(jnp.int32, shp, 2)
    b0 = lax.broadcasted_iota(jnp.int32, shp, 0)
    v = -yt_ref[...]
    ix = (b0 * 10000 + r2 * 128 + c1).astype(jnp.float32)
    w_ref[...] = jnp.stack([v, ix])

    q1 = lax.broadcasted_iota(jnp.int32, (128, 128), 0)
    q2 = lax.broadcasted_iota(jnp.int32, (128, 128), 1)

    def run(fn):
        def body(b, _):
            w_ref[:, b] = fn(w_ref[:, b])
            return 0
        lax.fori_loop(0, 8, body, 0)

    run(lambda wb: _stage0(wb, q1, q2))
    for k in range(8, 15):
        run(lambda wb, k=k: _stage_k(wb, q1, q2, k))

    wn = jnp.swapaxes(w_ref[...], 2, 3)
    idx_ref[...] = wn[1, :, :KROWS, :].astype(jnp.int32)
    scale_ref[...] = jax.nn.sigmoid(-wn[0, :, :KROWS, :])


KROWS = 20


def _topk(y_pad):
    B, n = y_pad.shape
    yt = y_pad.reshape(B, n // 128, 128).transpose(0, 2, 1)
    idx20, scale20 = pl.pallas_call(
        _sort_body,
        out_shape=(
            jax.ShapeDtypeStruct((B, KROWS, 128), jnp.int32),
            jax.ShapeDtypeStruct((B, KROWS, 128), jnp.float32),
        ),
        scratch_shapes=[pltpu.VMEM((2, B, 128, 128), jnp.float32)],
    )(yt)
    return idx20, scale20


def _make_gather(R, F, rows_per_w, chunk):
    info = plsc.get_sparse_core_info()
    nc, ns = info.num_cores, info.num_subcores
    nchunks = rows_per_w // chunk
    mesh = plsc.VectorSubcoreMesh(core_axis_name="c", subcore_axis_name="s")

    @functools.partial(
        pl.kernel,
        mesh=mesh,
        out_type=jax.ShapeDtypeStruct((R, F), jnp.float32),
        scratch_types=[
            pltpu.VMEM((chunk,), jnp.int32),
            pltpu.VMEM((chunk,), jnp.int32),
            pltpu.VMEM((chunk, F), jnp.float32),
            pltpu.VMEM((chunk, F), jnp.float32),
            pltpu.SemaphoreType.DMA,
            pltpu.SemaphoreType.DMA,
        ],
    )
    def gather(x_hbm, idx_hbm, out_hbm, i0, i1, r0, r1, s0, s1):
        wid = lax.axis_index("s") * nc + lax.axis_index("c")
        base = wid * rows_per_w
        bufs = ((i0, r0, s0), (i1, r1, s1))
        pltpu.sync_copy(idx_hbm.at[pl.ds(base, chunk)], i0)
        pend = pltpu.async_copy(x_hbm.at[i0], r0, s0)
        for t in range(nchunks):
            cur = pend
            if t + 1 < nchunks:
                iv, rv, sv = bufs[(t + 1) % 2]
                pltpu.sync_copy(
                    idx_hbm.at[pl.ds(base + (t + 1) * chunk, chunk)], iv)
                pend = pltpu.async_copy(x_hbm.at[iv], rv, sv)
            cur.wait()
            pltpu.sync_copy(bufs[t % 2][1],
                            out_hbm.at[pl.ds(base + t * chunk, chunk)])

    return gather


def _scale_body(g_ref, s_ref, o_ref):
    o_ref[...] = g_ref[:, :TOPK, :] * s_ref[0][:, :TOPK, None]


def _apply_scale(gathered, scale, kpad):
    B = scale.shape[0]
    F = gathered.shape[-1]
    g3 = gathered.reshape(B, kpad, F)
    s3 = scale.reshape(B, 1, kpad)
    return pl.pallas_call(
        _scale_body,
        grid=(B,),
        in_specs=[
            pl.BlockSpec((1, kpad, F), lambda b: (b, 0, 0)),
            pl.BlockSpec((1, 1, kpad), lambda b: (b, 0, 0)),
        ],
        out_specs=pl.BlockSpec((1, TOPK, F), lambda b: (b, 0, 0)),
        out_shape=jax.ShapeDtypeStruct((B, TOPK, F), jnp.float32),
    )(g3, s3)


def kernel(x, p):
    B, N, F = x.shape
    inv_norm = (1.0 / jnp.linalg.norm(p)).reshape(1, 1).astype(jnp.float32)

    y = _scores(x, p, inv_norm)
    y_pad = jnp.pad(y, ((0, 0), (0, NSORT - N)), constant_values=-jnp.inf)
    idx20, scale20 = _topk(y_pad)

    KPAD = KROWS * 128
    idx_flat = idx20.reshape(B * KPAD)

    RPAD = B * KPAD
    gather = _make_gather(RPAD, F, rows_per_w=RPAD // 32, chunk=128)
    gathered = gather(x.reshape(B * N, F), idx_flat)

    return _apply_scale(gathered, scale20, KPAD)
